# Initial kernel scaffold; baseline (speedup 1.0000x reference)
import jax
import jax.numpy as jnp
from jax.experimental import pallas as pl


def kernel(inputs, indices):
    raise NotImplementedError("write your pallas kernel here")


# speedup vs baseline: 1.6245x; 1.6245x over previous
import functools

import jax
import jax.numpy as jnp
from jax import lax
from jax.experimental import pallas as pl
from jax.experimental.pallas import tpu as pltpu
from jax.experimental.pallas import tpu_sc as plsc

_B = 4
_SEQ = 2048
_D = 1024
_ROWS = _B * _SEQ
_NC = 2
_NS = 16
_NW = _NC * _NS
_PER_W = _ROWS // _NW
_SEGS = _SEQ // _PER_W
_CH = 32
_NCHUNK = _PER_W // _CH


def _body(tab_hbm, idx_hbm, out_hbm, idx_v, buf_v, gsem, ssem):
    wid = lax.axis_index("s") * _NC + lax.axis_index("c")
    b = wid // _SEGS
    seg = wid % _SEGS

    pltpu.sync_copy(idx_hbm.at[pl.ds(seg * _NCHUNK, _NCHUNK)], idx_v)

    off = (b * _SEQ).astype(jnp.int32)
    for r in range(_NCHUNK):
        for k in range(_CH // 16):
            sl = (r, pl.ds(k * 16, 16))
            idx_v[sl] = idx_v[sl] + off

    base = wid * _PER_W

    def gather(c, nbuf):
        return pltpu.async_copy(tab_hbm.at[idx_v.at[c]], buf_v.at[nbuf], gsem)

    g = gather(0, 0)
    prev_s = None
    for c in range(_NCHUNK):
        g.wait()
        if prev_s is not None:
            prev_s.wait()
        if c + 1 < _NCHUNK:
            g = gather(c + 1, (c + 1) % 2)
        prev_s = pltpu.async_copy(
            buf_v.at[c % 2], out_hbm.at[pl.ds(base + c * _CH, _CH)], ssem
        )
    prev_s.wait()


@jax.jit
def _split_pathways(tab, idx):
    call = functools.partial(
        pl.kernel,
        out_type=jax.ShapeDtypeStruct((_ROWS, _D), jnp.float32),
        mesh=plsc.VectorSubcoreMesh(core_axis_name="c", subcore_axis_name="s"),
        scratch_types=[
            pltpu.VMEM((_NCHUNK, _CH), jnp.int32),
            pltpu.VMEM((2, _CH, _D), jnp.float32),
            pltpu.SemaphoreType.DMA,
            pltpu.SemaphoreType.DMA,
        ],
    )(_body)
    return call(tab, idx)


def kernel(inputs, indices):
    tab = inputs.reshape(_ROWS, _D)
    idx = indices.reshape(_SEGS * _NCHUNK, _CH)
    out = _split_pathways(tab, idx)
    return out.reshape(_B, _SEQ // 2, 2, _D)


# speedup vs baseline: 1.6757x; 1.0315x over previous
import functools

import jax
import jax.numpy as jnp
from jax import lax
from jax.experimental import pallas as pl
from jax.experimental.pallas import tpu as pltpu
from jax.experimental.pallas import tpu_sc as plsc

_B = 4
_SEQ = 2048
_D = 1024
_ROWS = _B * _SEQ
_NC = 2
_NS = 16
_NW = _NC * _NS
_PER_W = _ROWS // _NW
_SEGS = _SEQ // _PER_W
_CH = 32
_NCHUNK = _PER_W // _CH


def _body(tab_hbm, idx_hbm, out_hbm, idx_v, buf_v, gsem, ssem):
    wid = lax.axis_index("s") * _NC + lax.axis_index("c")
    b = wid // _SEGS
    seg = wid % _SEGS

    pltpu.sync_copy(idx_hbm.at[pl.ds(seg * _NCHUNK, _NCHUNK)], idx_v)

    off = (b * _SEQ).astype(jnp.int32)
    for r in range(_NCHUNK):
        for k in range(_CH // 16):
            sl = (r, pl.ds(k * 16, 16))
            idx_v[sl] = idx_v[sl] + off

    base = wid * _PER_W

    def gather(c, nbuf):
        return pltpu.async_copy(tab_hbm.at[idx_v.at[c]], buf_v.at[nbuf], gsem)

    gq = [gather(0, 0), gather(1, 1)]
    prev_s = None
    for c in range(_NCHUNK):
        gq.pop(0).wait()
        if prev_s is not None:
            prev_s.wait()
        if c + 2 < _NCHUNK:
            gq.append(gather(c + 2, (c + 2) % 3))
        prev_s = pltpu.async_copy(
            buf_v.at[c % 3], out_hbm.at[pl.ds(base + c * _CH, _CH)], ssem
        )
    prev_s.wait()


@jax.jit
def _split_pathways(tab, idx):
    call = functools.partial(
        pl.kernel,
        out_type=jax.ShapeDtypeStruct((_ROWS, _D), jnp.float32),
        mesh=plsc.VectorSubcoreMesh(core_axis_name="c", subcore_axis_name="s"),
        scratch_types=[
            pltpu.VMEM((_NCHUNK, _CH), jnp.int32),
            pltpu.VMEM((3, _CH, _D), jnp.float32),
            pltpu.SemaphoreType.DMA,
            pltpu.SemaphoreType.DMA,
        ],
    )(_body)
    return call(tab, idx)


def kernel(inputs, indices):
    tab = inputs.reshape(_ROWS, _D)
    idx = indices.reshape(_SEGS * _NCHUNK, _CH)
    out = _split_pathways(tab, idx)
    return out.reshape(_B, _SEQ // 2, 2, _D)


# speedup vs baseline: 3.0228x; 1.8039x over previous
import functools

import jax
import jax.numpy as jnp
from jax import lax
from jax.experimental import pallas as pl
from jax.experimental.pallas import tpu as pltpu
from jax.experimental.pallas import tpu_sc as plsc

_B = 4
_SEQ = 2048
_D = 1024
_NP = 2
_NI = 1024
_NC = 2
_NS = 16
_NW = _NC * _NS
_SEG = _NW // _B
_IBLK = _NI // _SEG
_SLAB = 16
_NCHUNK = _IBLK // _SLAB
_NBUF = 3


def _body(inp_hbm, idx_hbm, out_hbm, idx_v, buf_v, gsem, ssem):
    wid = lax.axis_index("s") * _NC + lax.axis_index("c")
    b = wid // _SEG
    i0 = (wid % _SEG) * _IBLK

    pltpu.sync_copy(idx_hbm.at[pl.ds(i0, _IBLK)], idx_v)

    tab = inp_hbm.at[b]

    def out_block(c):
        return out_hbm.at[b, pl.ds(i0 + c * _SLAB, _SLAB)]

    def drain(ref, sem):
        pltpu.make_async_copy(out_block(0), ref, sem).wait()

    for c in range(_NCHUNK):
        slot = c % _NBUF
        if c >= _NBUF:
            drain(buf_v.at[slot], ssem)

        def start_pair(j, carry):
            pltpu.async_copy(
                tab.at[idx_v.at[c * _SLAB + j]], buf_v.at[slot, j], gsem
            )
            return carry

        lax.fori_loop(0, _SLAB, start_pair, 0, unroll=4)
        drain(buf_v.at[slot], gsem)
        pltpu.async_copy(buf_v.at[slot], out_block(c), ssem)

    for slot in range(_NBUF):
        drain(buf_v.at[slot], ssem)


@jax.jit
def _split_pathways(inputs, indices):
    call = functools.partial(
        pl.kernel,
        out_type=jax.ShapeDtypeStruct((_B, _NI, _NP, _D), jnp.float32),
        mesh=plsc.VectorSubcoreMesh(core_axis_name="c", subcore_axis_name="s"),
        scratch_types=[
            pltpu.VMEM((_IBLK, _NP), jnp.int32),
            pltpu.VMEM((_NBUF, _SLAB, _NP, _D), jnp.float32),
            pltpu.SemaphoreType.DMA,
            pltpu.SemaphoreType.DMA,
        ],
    )(_body)
    return call(inputs, indices)


def kernel(inputs, indices):
    return _split_pathways(inputs, indices)


# speedup vs baseline: 3.1413x; 1.0392x over previous
import functools

import jax
import jax.numpy as jnp
from jax import lax
from jax.experimental import pallas as pl
from jax.experimental.pallas import tpu as pltpu
from jax.experimental.pallas import tpu_sc as plsc

_B = 4
_SEQ = 2048
_D = 1024
_NP = 2
_NI = 1024
_NC = 2
_NS = 16
_NW = _NC * _NS
_SEG = _NW // _B
_IBLK = _NI // _SEG
_SLAB = 16
_NCHUNK = _IBLK // _SLAB
_NBUF = 3


def _body(inp_hbm, idx_hbm, out_hbm, idx_v, buf_v, gsem, ssem):
    wid = lax.axis_index("s") * _NC + lax.axis_index("c")
    b = wid // _SEG
    i0 = (wid % _SEG) * _IBLK

    pltpu.sync_copy(idx_hbm.at[pl.ds(i0, _IBLK)], idx_v)

    tab = inp_hbm.at[b]

    def out_block(c):
        return out_hbm.at[b, pl.ds(i0 + c * _SLAB, _SLAB)]

    def drain(ref, sem):
        pltpu.make_async_copy(out_block(0), ref, sem).wait()

    def start_gathers(c):
        slot = c % _NBUF

        def start_pair(j, carry):
            pltpu.async_copy(
                tab.at[idx_v.at[c * _SLAB + j]], buf_v.at[slot, j], gsem
            )
            return carry

        lax.fori_loop(0, _SLAB, start_pair, 0, unroll=8)

    start_gathers(0)
    for c in range(_NCHUNK):
        if c + 1 < _NCHUNK:
            if c + 1 >= _NBUF:
                drain(buf_v.at[(c + 1) % _NBUF], ssem)
            start_gathers(c + 1)
        drain(buf_v.at[c % _NBUF], gsem)
        pltpu.async_copy(buf_v.at[c % _NBUF], out_block(c), ssem)

    for slot in range(_NBUF):
        drain(buf_v.at[slot], ssem)


@jax.jit
def _split_pathways(inputs, indices):
    call = functools.partial(
        pl.kernel,
        out_type=jax.ShapeDtypeStruct((_B, _NI, _NP, _D), jnp.float32),
        mesh=plsc.VectorSubcoreMesh(core_axis_name="c", subcore_axis_name="s"),
        scratch_types=[
            pltpu.VMEM((_IBLK, _NP), jnp.int32),
            pltpu.VMEM((_NBUF, _SLAB, _NP, _D), jnp.float32),
            pltpu.SemaphoreType.DMA,
            pltpu.SemaphoreType.DMA,
        ],
    )(_body)
    return call(inputs, indices)


def kernel(inputs, indices):
    return _split_pathways(inputs, indices)
